# Initial kernel scaffold; baseline (speedup 1.0000x reference)
import jax
import jax.numpy as jnp
from jax.experimental import pallas as pl


def kernel(x):
    raise NotImplementedError("write your pallas kernel here")


# speedup vs baseline: 126.1373x; 126.1373x over previous
import functools

import jax
import jax.numpy as jnp
import numpy as np
from jax.experimental import pallas as pl

N = 65536
F = 256
Q = 100

_LO_IDX = (np.arange(Q, dtype=np.int64) * (N - 1)) // (Q - 1)

_ROWS_PER_BLOCK = 1024


def _bin_body(b_ref, x_ref, o_ref):
    x = x_ref[...]
    acc = jnp.zeros_like(x)
    for i in range(Q):
        acc = acc + jnp.where(b_ref[i, :][None, :] < x, 1.0, 0.0).astype(jnp.float32)
    o_ref[...] = acc / jnp.float32(Q)


def _bin_count(boundaries, x):
    grid = (N // _ROWS_PER_BLOCK,)
    return pl.pallas_call(
        _bin_body,
        grid=grid,
        in_specs=[
            pl.BlockSpec((Q, F), lambda i: (0, 0)),
            pl.BlockSpec((_ROWS_PER_BLOCK, F), lambda i: (i, 0)),
        ],
        out_specs=pl.BlockSpec((_ROWS_PER_BLOCK, F), lambda i: (i, 0)),
        out_shape=jax.ShapeDtypeStruct((N, F), jnp.float32),
    )(boundaries, x)


def kernel(x):
    s = jnp.sort(x, axis=0)
    boundaries = s[_LO_IDX, :]
    return _bin_count(boundaries, x)


# speedup vs baseline: 378.3795x; 2.9997x over previous
import functools

import jax
import jax.numpy as jnp
import numpy as np
from jax import lax
from jax.experimental import pallas as pl
from jax.experimental.pallas import tpu as pltpu
from jax.experimental.pallas import tpu_sc as plsc

N = 65536
F = 256
Q = 100

NC, NS = 2, 16
NW = NC * NS
CPT = F // NW
LANES = 16

K = 4096
S = 32
CH = 4096
NCHUNK = N // CH
UNR = 8
HKW = CPT * K
SIDN = CPT * Q
SUBW = SIDN * S
BIGW = max(2 * HKW, HKW + 2 * SUBW)

_ROWS_PER_BLOCK = 1024
_MM_ROWS = 2048


def _sc_body(xt, mnmx, out, buf, big, recs, mm, bnd, sem):
    wid = lax.axis_index("s") * NC + lax.axis_index("c")
    row0 = wid * CPT
    iota = lax.iota(jnp.int32, LANES)
    col16 = jnp.bitwise_and(iota, CPT - 1)
    slot16 = lax.shift_right_logical(iota, 3)
    ones_i = jnp.ones((LANES,), jnp.int32)
    zeros_i = jnp.zeros((LANES,), jnp.int32)
    lane0 = iota == 0

    def sload(ref, idx):
        return ref[pl.ds(idx, LANES)][0]

    def sstore(ref, idx, val):
        plsc.store_scatter(
            ref,
            [jnp.broadcast_to(idx, (LANES,))],
            jnp.broadcast_to(val, (LANES,)),
            mask=lane0,
        )

    def load_chunk(cc):
        cps = [
            pltpu.async_copy(
                xt.at[pl.ds((row0 + r) * N + cc * CH, CH)],
                buf.at[pl.ds(r * CH, CH)],
                sem,
            )
            for r in range(CPT)
        ]
        for cp in cps:
            cp.wait()

    pltpu.sync_copy(mnmx.at[pl.ds(row0, CPT)], mm.at[pl.ds(0, CPT)])
    pltpu.sync_copy(mnmx.at[pl.ds(F + row0, CPT)], mm.at[pl.ds(CPT, CPT)])
    cmin = plsc.load_gather(mm, [col16])
    cmax = plsc.load_gather(mm, [col16 + CPT])
    w = jnp.maximum(cmax - cmin, jnp.float32(1e-30))
    invw = jnp.float32(K) / w
    wb = w * jnp.float32(1.0 / K)
    invw2 = invw * jnp.float32(S)
    mm[pl.ds(16, 16)] = cmin
    mm[pl.ds(32, 16)] = wb

    def zero_region(base, nwords, fill):
        def z(i, _):
            big[pl.ds(base + i * 16, 16)] = fill
            return 0

        lax.fori_loop(0, nwords // 16, z, 0)

    zero_region(0, 2 * HKW, zeros_i)
    histbase = slot16 * HKW + col16 * K
    bufbase = col16 * CH + slot16

    def h_chunk(cc, _):
        load_chunk(cc)

        def it(p, __):
            for u in range(UNR):
                pos = p * (2 * UNR) + 2 * u
                v = plsc.load_gather(buf, [bufbase + pos])
                bi = jnp.minimum(((v - cmin) * invw).astype(jnp.int32), K - 1)
                plsc.addupdate_scatter(big, [histbase + bi], ones_i)
            return 0

        lax.fori_loop(0, CH // (2 * UNR), it, 0)
        return 0

    lax.fori_loop(0, NCHUNK, h_chunk, 0)

    def mrg1(i, _):
        big[pl.ds(i * 16, 16)] = big[pl.ds(i * 16, 16)] + big[pl.ds(HKW + i * 16, 16)]
        return 0

    lax.fori_loop(0, HKW // 16, mrg1, 0)

    def cdf_col(c, _):
        def cdf_t(t, carry):
            bptr, cum, h = carry
            rt = (t * (N - 1)) // (Q - 1)

            def wcond(st):
                b2, c2, h2 = st
                return c2 + h2 <= rt

            def wbody(st):
                b2, c2, h2 = st
                return b2 + 1, c2 + h2, sload(big, c * K + b2 + 1)

            bptr, cum, h = lax.while_loop(wcond, wbody, (bptr, cum, h))
            sstore(recs, c * Q + t, bptr)
            sstore(recs, SIDN + c * Q + t, rt - cum)
            return bptr, cum, h

        h0 = sload(big, c * K)
        lax.fori_loop(0, Q, cdf_t, (jnp.int32(0), jnp.int32(0), h0))
        return 0

    lax.fori_loop(0, CPT, cdf_col, 0)

    neg16 = jnp.full((LANES,), -1, jnp.int32)
    zero_region(0, HKW, neg16)
    zero_region(HKW, 2 * SUBW, zeros_i)

    def bld_c(c, _):
        def bld_t(t, __):
            addr = c * K + sload(recs, c * Q + t)
            cur = sload(big, addr)
            sstore(big, addr, jnp.where(cur < 0, c * Q + t, cur))
            return 0

        lax.fori_loop(0, Q, bld_t, 0)
        return 0

    lax.fori_loop(0, CPT, bld_c, 0)

    colK = col16 * K
    sub_base = HKW + slot16 * SUBW

    def s_chunk(cc, _):
        load_chunk(cc)

        def it(p, __):
            for u in range(UNR):
                pos = p * (2 * UNR) + 2 * u
                v = plsc.load_gather(buf, [bufbase + pos])
                bi = jnp.minimum(((v - cmin) * invw).astype(jnp.int32), K - 1)
                sid = plsc.load_gather(big, [colK + bi])
                msk = sid >= 0
                sidc = jnp.maximum(sid, 0)
                lo1 = cmin + bi.astype(jnp.float32) * wb
                sub = jnp.clip(((v - lo1) * invw2).astype(jnp.int32), 0, S - 1)
                plsc.addupdate_scatter(
                    big, [sub_base + sidc * S + sub], ones_i, mask=msk
                )
            return 0

        lax.fori_loop(0, CH // (2 * UNR), it, 0)
        return 0

    lax.fori_loop(0, NCHUNK, s_chunk, 0)

    def mrg2(i, _):
        big[pl.ds(HKW + i * 16, 16)] = (
            big[pl.ds(HKW + i * 16, 16)] + big[pl.ds(HKW + SUBW + i * 16, 16)]
        )
        return 0

    lax.fori_loop(0, SUBW // 16, mrg2, 0)

    def fin_c(c, _):
        mn_c = sload(mm, 16 + c)
        wb_c = sload(mm, 32 + c)
        wb32_c = wb_c * jnp.float32(1.0 / S)

        def fin_t(t, __):
            bkt = sload(recs, c * Q + t)
            j = sload(recs, SIDN + c * Q + t)
            sid = sload(big, c * K + bkt)
            sb = HKW + sid * S

            def wcond(st):
                s2, c2, h2 = st
                return c2 + h2 <= j

            def wbody(st):
                s2, c2, h2 = st
                return s2 + 1, c2 + h2, sload(big, sb + s2 + 1)

            sptr, _, _ = lax.while_loop(
                wcond, wbody, (jnp.int32(0), jnp.int32(0), sload(big, sb))
            )
            e = mn_c + bkt.astype(jnp.float32) * wb_c + sptr.astype(jnp.float32) * wb32_c
            sstore(bnd, c * Q + t, e)
            return 0

        lax.fori_loop(0, Q, fin_t, 0)
        return 0

    lax.fori_loop(0, CPT, fin_c, 0)

    pltpu.sync_copy(bnd, out.at[pl.ds(row0 * Q, CPT * Q)])


_sc_select = functools.partial(
    pl.kernel,
    out_type=jax.ShapeDtypeStruct((F * Q,), jnp.float32),
    mesh=plsc.VectorSubcoreMesh(
        core_axis_name="c", subcore_axis_name="s", num_cores=NC, num_subcores=NS
    ),
    compiler_params=pltpu.CompilerParams(needs_layout_passes=False),
    scratch_types=[
        pltpu.VMEM((CPT * CH,), jnp.float32),
        pltpu.VMEM((BIGW + LANES,), jnp.int32),
        pltpu.VMEM((2 * SIDN + LANES,), jnp.int32),
        pltpu.VMEM((80,), jnp.float32),
        pltpu.VMEM((SIDN,), jnp.float32),
        pltpu.SemaphoreType.DMA,
    ],
)(_sc_body)


def _mm_body(x_ref, o_ref):
    i = pl.program_id(0)
    xb = x_ref[...]
    mn = jnp.min(xb, axis=0)
    mx = jnp.max(xb, axis=0)

    @pl.when(i == 0)
    def _init():
        o_ref[0, :] = mn
        o_ref[8, :] = mx

    @pl.when(i > 0)
    def _acc():
        o_ref[0, :] = jnp.minimum(o_ref[0, :], mn)
        o_ref[8, :] = jnp.maximum(o_ref[8, :], mx)


def _col_minmax(x):
    out = pl.pallas_call(
        _mm_body,
        grid=(N // _MM_ROWS,),
        in_specs=[pl.BlockSpec((_MM_ROWS, F), lambda i: (i, 0))],
        out_specs=pl.BlockSpec((16, F), lambda i: (0, 0)),
        out_shape=jax.ShapeDtypeStruct((16, F), jnp.float32),
    )(x)
    return jnp.concatenate([out[0], out[8]])


def _bin_body(b_ref, x_ref, o_ref):
    x = x_ref[...]
    acc = jnp.zeros_like(x)
    for i in range(Q):
        acc = acc + jnp.where(b_ref[i, :][None, :] < x, 1.0, 0.0).astype(jnp.float32)
    o_ref[...] = acc / jnp.float32(Q)


def _bin_count(boundaries, x):
    grid = (N // _ROWS_PER_BLOCK,)
    return pl.pallas_call(
        _bin_body,
        grid=grid,
        in_specs=[
            pl.BlockSpec((Q, F), lambda i: (0, 0)),
            pl.BlockSpec((_ROWS_PER_BLOCK, F), lambda i: (i, 0)),
        ],
        out_specs=pl.BlockSpec((_ROWS_PER_BLOCK, F), lambda i: (i, 0)),
        out_shape=jax.ShapeDtypeStruct((N, F), jnp.float32),
    )(boundaries, x)


def kernel(x):
    xt = x.T.reshape(-1)
    mnmx = _col_minmax(x)
    bnd = _sc_select(xt, mnmx).reshape(F, Q)
    return _bin_count(bnd.T, x)


# speedup vs baseline: 589.3371x; 1.5575x over previous
import functools

import jax
import jax.numpy as jnp
import numpy as np
from jax import lax
from jax.experimental import pallas as pl
from jax.experimental.pallas import tpu as pltpu
from jax.experimental.pallas import tpu_sc as plsc

N = 65536
F = 256
Q = 100

NC, NS = 2, 16
NW = NC * NS
CPT = F // NW
LANES = 16

K = 4096
S = 32
CH = 4096
NCHUNK = N // CH
UNR = 8
HKW = CPT * K
SIDN = CPT * Q
SUBW = SIDN * S
BIGW = max(2 * HKW, HKW + 2 * SUBW)

_ROWS_PER_BLOCK = 1024
_MM_ROWS = 2048


def _sc_body(xt, mnmx, out, buf, big, recs, mm, bnd, sem):
    wid = lax.axis_index("s") * NC + lax.axis_index("c")
    row0 = wid * CPT
    iota = lax.iota(jnp.int32, LANES)
    col16 = jnp.bitwise_and(iota, CPT - 1)
    slot16 = lax.shift_right_logical(iota, 3)
    ones_i = jnp.ones((LANES,), jnp.int32)
    zeros_i = jnp.zeros((LANES,), jnp.int32)
    lane0 = iota == 0

    def sload(ref, idx):
        return ref[pl.ds(idx, LANES)][0]

    def sstore(ref, idx, val):
        plsc.store_scatter(
            ref,
            [jnp.broadcast_to(idx, (LANES,))],
            jnp.broadcast_to(val, (LANES,)),
            mask=lane0,
        )

    def load_chunk(cc):
        cps = [
            pltpu.async_copy(
                xt.at[pl.ds((row0 + r) * N + cc * CH, CH)],
                buf.at[pl.ds(r * CH, CH)],
                sem,
            )
            for r in range(CPT)
        ]
        for cp in cps:
            cp.wait()

    pltpu.sync_copy(mnmx.at[pl.ds(row0, CPT)], mm.at[pl.ds(0, CPT)])
    pltpu.sync_copy(mnmx.at[pl.ds(F + row0, CPT)], mm.at[pl.ds(CPT, CPT)])
    cmin = plsc.load_gather(mm, [col16])
    cmax = plsc.load_gather(mm, [col16 + CPT])
    w = jnp.maximum(cmax - cmin, jnp.float32(1e-30))
    invw = jnp.float32(K) / w
    wb = w * jnp.float32(1.0 / K)
    invw2 = invw * jnp.float32(S)
    mm[pl.ds(16, 16)] = cmin
    mm[pl.ds(32, 16)] = wb

    def zero_region(base, nwords, fill):
        def z(i, _):
            big[pl.ds(base + i * 16, 16)] = fill
            return 0

        lax.fori_loop(0, nwords // 16, z, 0)

    zero_region(0, 2 * HKW, zeros_i)
    histbase = slot16 * HKW + col16 * K
    bufbase = col16 * CH + slot16

    def h_chunk(cc, _):
        load_chunk(cc)

        @plsc.parallel_loop(0, CH // 2, unroll=UNR)
        def _h_it(p):
            v = plsc.load_gather(buf, [bufbase + p * 2])
            bi = jnp.minimum(((v - cmin) * invw).astype(jnp.int32), K - 1)
            plsc.addupdate_scatter(big, [histbase + bi], ones_i)

        return 0

    lax.fori_loop(0, NCHUNK, h_chunk, 0)

    def mrg1(i, _):
        big[pl.ds(i * 16, 16)] = big[pl.ds(i * 16, 16)] + big[pl.ds(HKW + i * 16, 16)]
        return 0

    lax.fori_loop(0, HKW // 16, mrg1, 0)

    def cdf_col(c, _):
        def cdf_t(t, carry):
            bptr, cum, h = carry
            rt = (t * (N - 1)) // (Q - 1)

            def wcond(st):
                b2, c2, h2 = st
                return c2 + h2 <= rt

            def wbody(st):
                b2, c2, h2 = st
                return b2 + 1, c2 + h2, sload(big, c * K + b2 + 1)

            bptr, cum, h = lax.while_loop(wcond, wbody, (bptr, cum, h))
            sstore(recs, c * Q + t, bptr)
            sstore(recs, SIDN + c * Q + t, rt - cum)
            return bptr, cum, h

        h0 = sload(big, c * K)
        lax.fori_loop(0, Q, cdf_t, (jnp.int32(0), jnp.int32(0), h0))
        return 0

    lax.fori_loop(0, CPT, cdf_col, 0)

    neg16 = jnp.full((LANES,), -1, jnp.int32)
    zero_region(0, HKW, neg16)
    zero_region(HKW, 2 * SUBW, zeros_i)

    def bld_c(c, _):
        def bld_t(t, __):
            addr = c * K + sload(recs, c * Q + t)
            cur = sload(big, addr)
            sstore(big, addr, jnp.where(cur < 0, c * Q + t, cur))
            return 0

        lax.fori_loop(0, Q, bld_t, 0)
        return 0

    lax.fori_loop(0, CPT, bld_c, 0)

    colK = col16 * K
    sub_base = HKW + slot16 * SUBW

    def s_chunk(cc, _):
        load_chunk(cc)

        @plsc.parallel_loop(0, CH // 2, unroll=UNR)
        def _s_it(p):
            v = plsc.load_gather(buf, [bufbase + p * 2])
            bi = jnp.minimum(((v - cmin) * invw).astype(jnp.int32), K - 1)
            sid = plsc.load_gather(big, [colK + bi])
            msk = sid >= 0
            sidc = jnp.maximum(sid, 0)
            lo1 = cmin + bi.astype(jnp.float32) * wb
            sub = jnp.clip(((v - lo1) * invw2).astype(jnp.int32), 0, S - 1)
            plsc.addupdate_scatter(big, [sub_base + sidc * S + sub], ones_i, mask=msk)

        return 0

    lax.fori_loop(0, NCHUNK, s_chunk, 0)

    def mrg2(i, _):
        big[pl.ds(HKW + i * 16, 16)] = (
            big[pl.ds(HKW + i * 16, 16)] + big[pl.ds(HKW + SUBW + i * 16, 16)]
        )
        return 0

    lax.fori_loop(0, SUBW // 16, mrg2, 0)

    def fin_c(c, _):
        mn_c = sload(mm, 16 + c)
        wb_c = sload(mm, 32 + c)
        wb32_c = wb_c * jnp.float32(1.0 / S)

        def fin_t(t, __):
            bkt = sload(recs, c * Q + t)
            j = sload(recs, SIDN + c * Q + t)
            sid = sload(big, c * K + bkt)
            sb = HKW + sid * S

            def wcond(st):
                s2, c2, h2 = st
                return c2 + h2 <= j

            def wbody(st):
                s2, c2, h2 = st
                return s2 + 1, c2 + h2, sload(big, sb + s2 + 1)

            sptr, _, _ = lax.while_loop(
                wcond, wbody, (jnp.int32(0), jnp.int32(0), sload(big, sb))
            )
            e = mn_c + bkt.astype(jnp.float32) * wb_c + sptr.astype(jnp.float32) * wb32_c
            sstore(bnd, c * Q + t, e)
            return 0

        lax.fori_loop(0, Q, fin_t, 0)
        return 0

    lax.fori_loop(0, CPT, fin_c, 0)

    pltpu.sync_copy(bnd, out.at[pl.ds(row0 * Q, CPT * Q)])


_sc_select = functools.partial(
    pl.kernel,
    out_type=jax.ShapeDtypeStruct((F * Q,), jnp.float32),
    mesh=plsc.VectorSubcoreMesh(
        core_axis_name="c", subcore_axis_name="s", num_cores=NC, num_subcores=NS
    ),
    compiler_params=pltpu.CompilerParams(needs_layout_passes=False),
    scratch_types=[
        pltpu.VMEM((CPT * CH,), jnp.float32),
        pltpu.VMEM((BIGW + LANES,), jnp.int32),
        pltpu.VMEM((2 * SIDN + LANES,), jnp.int32),
        pltpu.VMEM((80,), jnp.float32),
        pltpu.VMEM((SIDN,), jnp.float32),
        pltpu.SemaphoreType.DMA,
    ],
)(_sc_body)


def _mm_body(x_ref, o_ref):
    i = pl.program_id(0)
    xb = x_ref[...]
    mn = jnp.min(xb, axis=0)
    mx = jnp.max(xb, axis=0)

    @pl.when(i == 0)
    def _init():
        o_ref[0, :] = mn
        o_ref[8, :] = mx

    @pl.when(i > 0)
    def _acc():
        o_ref[0, :] = jnp.minimum(o_ref[0, :], mn)
        o_ref[8, :] = jnp.maximum(o_ref[8, :], mx)


def _col_minmax(x):
    out = pl.pallas_call(
        _mm_body,
        grid=(N // _MM_ROWS,),
        in_specs=[pl.BlockSpec((_MM_ROWS, F), lambda i: (i, 0))],
        out_specs=pl.BlockSpec((16, F), lambda i: (0, 0)),
        out_shape=jax.ShapeDtypeStruct((16, F), jnp.float32),
    )(x)
    return jnp.concatenate([out[0], out[8]])


def _bin_body(b_ref, x_ref, o_ref):
    x = x_ref[...]
    acc = jnp.zeros_like(x)
    for i in range(Q):
        acc = acc + jnp.where(b_ref[i, :][None, :] < x, 1.0, 0.0).astype(jnp.float32)
    o_ref[...] = acc / jnp.float32(Q)


def _bin_count(boundaries, x):
    grid = (N // _ROWS_PER_BLOCK,)
    return pl.pallas_call(
        _bin_body,
        grid=grid,
        in_specs=[
            pl.BlockSpec((Q, F), lambda i: (0, 0)),
            pl.BlockSpec((_ROWS_PER_BLOCK, F), lambda i: (i, 0)),
        ],
        out_specs=pl.BlockSpec((_ROWS_PER_BLOCK, F), lambda i: (i, 0)),
        out_shape=jax.ShapeDtypeStruct((N, F), jnp.float32),
    )(boundaries, x)


def kernel(x):
    xt = x.T.reshape(-1)
    mnmx = _col_minmax(x)
    bnd = _sc_select(xt, mnmx).reshape(F, Q)
    return _bin_count(bnd.T, x)


# speedup vs baseline: 840.6377x; 1.4264x over previous
import functools

import jax
import jax.numpy as jnp
import numpy as np
from jax import lax
from jax.experimental import pallas as pl
from jax.experimental.pallas import tpu as pltpu
from jax.experimental.pallas import tpu_sc as plsc

N = 65536
F = 256
Q = 100

NC, NS = 2, 16
NW = NC * NS
CPT = F // NW
LANES = 16

K = 4096
S = 32
CH = 4096
NCHUNK = N // CH
UNR = 8
HKW = CPT * K
SIDN = CPT * Q
SUBW = SIDN * S
BIGW = max(2 * HKW, HKW + 2 * SUBW)

_ROWS_PER_BLOCK = 1024
_MM_ROWS = 2048


def _sc_body(xt, mnmx, out, buf, big, recs, mm, bnd, sem):
    wid = lax.axis_index("s") * NC + lax.axis_index("c")
    row0 = wid * CPT
    iota = lax.iota(jnp.int32, LANES)
    col16 = jnp.bitwise_and(iota, CPT - 1)
    slot16 = lax.shift_right_logical(iota, 3)
    ones_i = jnp.ones((LANES,), jnp.int32)
    zeros_i = jnp.zeros((LANES,), jnp.int32)
    lane0 = iota == 0

    def sload(ref, idx):
        return ref[pl.ds(idx, LANES)][0]

    def sstore(ref, idx, val):
        plsc.store_scatter(
            ref,
            [jnp.broadcast_to(idx, (LANES,))],
            jnp.broadcast_to(val, (LANES,)),
            mask=lane0,
        )

    def load_chunk(cc):
        cps = [
            pltpu.async_copy(
                xt.at[pl.ds((row0 + r) * N + cc * CH, CH)],
                buf.at[pl.ds(r * CH, CH)],
                sem,
            )
            for r in range(CPT)
        ]
        for cp in cps:
            cp.wait()

    pltpu.sync_copy(mnmx.at[pl.ds(row0, CPT)], mm.at[pl.ds(0, CPT)])
    pltpu.sync_copy(mnmx.at[pl.ds(F + row0, CPT)], mm.at[pl.ds(CPT, CPT)])
    cmin = plsc.load_gather(mm, [col16])
    cmax = plsc.load_gather(mm, [col16 + CPT])
    w = jnp.maximum(cmax - cmin, jnp.float32(1e-30))
    invw = jnp.float32(K) / w
    wb = w * jnp.float32(1.0 / K)
    invw2 = invw * jnp.float32(S)
    mm[pl.ds(16, 16)] = cmin
    mm[pl.ds(32, 16)] = wb

    def zero_region(base, nwords, fill):
        def z(i, _):
            big[pl.ds(base + i * 16, 16)] = fill
            return 0

        lax.fori_loop(0, nwords // 16, z, 0)

    zero_region(0, 2 * HKW, zeros_i)
    histbase = slot16 * HKW + col16 * K
    bufbase = col16 * CH + slot16

    def h_chunk(cc, _):
        load_chunk(cc)

        @plsc.parallel_loop(0, CH // 2, unroll=UNR)
        def _h_it(p):
            v = plsc.load_gather(buf, [bufbase + p * 2])
            bi = jnp.minimum(((v - cmin) * invw).astype(jnp.int32), K - 1)
            plsc.addupdate_scatter(big, [histbase + bi], ones_i)

        return 0

    lax.fori_loop(0, NCHUNK, h_chunk, 0)

    def mrg1(i, _):
        big[pl.ds(i * 16, 16)] = big[pl.ds(i * 16, 16)] + big[pl.ds(HKW + i * 16, 16)]
        return 0

    lax.fori_loop(0, HKW // 16, mrg1, 0)

    @plsc.parallel_loop(0, HKW // 16, unroll=4)
    def _cs(i):
        sstore(big, HKW + i, jnp.sum(big[pl.ds(i * 16, 16)]))

    def cdf_col(c, _):
        csbase = HKW + c * (K // 16)

        def cdf_t(t, carry):
            ccp, cumc, hc = carry
            rt = (t * (N - 1)) // (Q - 1)

            def c_cond(st):
                c2, cm2, h2 = st
                return cm2 + h2 <= rt

            def c_body(st):
                c2, cm2, h2 = st
                return c2 + 1, cm2 + h2, sload(big, csbase + c2 + 1)

            ccp, cumc, hc = lax.while_loop(c_cond, c_body, (ccp, cumc, hc))
            fb = c * K + ccp * 16

            def f_cond(st):
                b2, cm2, h2 = st
                return cm2 + h2 <= rt

            def f_body(st):
                b2, cm2, h2 = st
                return b2 + 1, cm2 + h2, sload(big, fb + b2 + 1)

            bptr, cumf, _ = lax.while_loop(
                f_cond, f_body, (jnp.int32(0), cumc, sload(big, fb))
            )
            sstore(recs, c * Q + t, ccp * 16 + bptr)
            sstore(recs, SIDN + c * Q + t, rt - cumf)
            return ccp, cumc, hc

        h0 = sload(big, csbase)
        lax.fori_loop(0, Q, cdf_t, (jnp.int32(0), jnp.int32(0), h0))
        return 0

    lax.fori_loop(0, CPT, cdf_col, 0)

    neg16 = jnp.full((LANES,), -1, jnp.int32)
    zero_region(0, HKW, neg16)
    zero_region(HKW, 2 * SUBW, zeros_i)

    def bld_c(c, _):
        def bld_t(t, __):
            addr = c * K + sload(recs, c * Q + t)
            cur = sload(big, addr)
            sstore(big, addr, jnp.where(cur < 0, c * Q + t, cur))
            return 0

        lax.fori_loop(0, Q, bld_t, 0)
        return 0

    lax.fori_loop(0, CPT, bld_c, 0)

    colK = col16 * K
    sub_base = HKW + slot16 * SUBW

    def s_chunk(cc, _):
        load_chunk(cc)

        @plsc.parallel_loop(0, CH // 2, unroll=UNR)
        def _s_it(p):
            v = plsc.load_gather(buf, [bufbase + p * 2])
            bi = jnp.minimum(((v - cmin) * invw).astype(jnp.int32), K - 1)
            sid = plsc.load_gather(big, [colK + bi])
            msk = sid >= 0
            sidc = jnp.maximum(sid, 0)
            lo1 = cmin + bi.astype(jnp.float32) * wb
            sub = jnp.clip(((v - lo1) * invw2).astype(jnp.int32), 0, S - 1)
            plsc.addupdate_scatter(big, [sub_base + sidc * S + sub], ones_i, mask=msk)

        return 0

    lax.fori_loop(0, NCHUNK, s_chunk, 0)

    def mrg2(i, _):
        big[pl.ds(HKW + i * 16, 16)] = (
            big[pl.ds(HKW + i * 16, 16)] + big[pl.ds(HKW + SUBW + i * 16, 16)]
        )
        return 0

    lax.fori_loop(0, SUBW // 16, mrg2, 0)

    def fin_c(c, _):
        mn_c = sload(mm, 16 + c)
        wb_c = sload(mm, 32 + c)
        wb32_c = wb_c * jnp.float32(1.0 / S)

        def fin_t(t, __):
            bkt = sload(recs, c * Q + t)
            j = sload(recs, SIDN + c * Q + t)
            sid = sload(big, c * K + bkt)
            sb = HKW + sid * S
            p0 = plsc.cumsum(big[pl.ds(sb, 16)])
            p1 = plsc.cumsum(big[pl.ds(sb + 16, 16)]) + p0[15]
            c0 = plsc.all_reduce_population_count(p0 <= j)[0]
            c1 = plsc.all_reduce_population_count(p1 <= j)[0]
            sptr = c0 + c1
            e = mn_c + bkt.astype(jnp.float32) * wb_c + sptr.astype(jnp.float32) * wb32_c
            sstore(bnd, c * Q + t, e)
            return 0

        lax.fori_loop(0, Q, fin_t, 0)
        return 0

    lax.fori_loop(0, CPT, fin_c, 0)

    pltpu.sync_copy(bnd, out.at[pl.ds(row0 * Q, CPT * Q)])


_sc_select = functools.partial(
    pl.kernel,
    out_type=jax.ShapeDtypeStruct((F * Q,), jnp.float32),
    mesh=plsc.VectorSubcoreMesh(
        core_axis_name="c", subcore_axis_name="s", num_cores=NC, num_subcores=NS
    ),
    compiler_params=pltpu.CompilerParams(needs_layout_passes=False),
    scratch_types=[
        pltpu.VMEM((CPT * CH,), jnp.float32),
        pltpu.VMEM((BIGW + LANES,), jnp.int32),
        pltpu.VMEM((2 * SIDN + LANES,), jnp.int32),
        pltpu.VMEM((80,), jnp.float32),
        pltpu.VMEM((SIDN,), jnp.float32),
        pltpu.SemaphoreType.DMA,
    ],
)(_sc_body)


def _mm_body(x_ref, o_ref):
    i = pl.program_id(0)
    xb = x_ref[...]
    mn = jnp.min(xb, axis=0)
    mx = jnp.max(xb, axis=0)

    @pl.when(i == 0)
    def _init():
        o_ref[0, :] = mn
        o_ref[8, :] = mx

    @pl.when(i > 0)
    def _acc():
        o_ref[0, :] = jnp.minimum(o_ref[0, :], mn)
        o_ref[8, :] = jnp.maximum(o_ref[8, :], mx)


def _col_minmax(x):
    out = pl.pallas_call(
        _mm_body,
        grid=(N // _MM_ROWS,),
        in_specs=[pl.BlockSpec((_MM_ROWS, F), lambda i: (i, 0))],
        out_specs=pl.BlockSpec((16, F), lambda i: (0, 0)),
        out_shape=jax.ShapeDtypeStruct((16, F), jnp.float32),
    )(x)
    return jnp.concatenate([out[0], out[8]])


def _bin_body(b_ref, x_ref, o_ref):
    x = x_ref[...]
    acc = jnp.zeros_like(x)
    for i in range(Q):
        acc = acc + jnp.where(b_ref[i, :][None, :] < x, 1.0, 0.0).astype(jnp.float32)
    o_ref[...] = acc / jnp.float32(Q)


def _bin_count(boundaries, x):
    grid = (N // _ROWS_PER_BLOCK,)
    return pl.pallas_call(
        _bin_body,
        grid=grid,
        in_specs=[
            pl.BlockSpec((Q, F), lambda i: (0, 0)),
            pl.BlockSpec((_ROWS_PER_BLOCK, F), lambda i: (i, 0)),
        ],
        out_specs=pl.BlockSpec((_ROWS_PER_BLOCK, F), lambda i: (i, 0)),
        out_shape=jax.ShapeDtypeStruct((N, F), jnp.float32),
    )(boundaries, x)


def kernel(x):
    xt = x.T.reshape(-1)
    mnmx = _col_minmax(x)
    bnd = _sc_select(xt, mnmx).reshape(F, Q)
    return _bin_count(bnd.T, x)
